# baseline (speedup 1.0000x reference)
import functools

import numpy as np

import jax
import jax.numpy as jnp
from jax import lax
from jax.experimental import pallas as pl
from jax.experimental.pallas import tpu as pltpu
from jax.experimental.pallas import tpu_sc as plsc

NC = 2
NS = 16
NW = NC * NS

NSEG = 512
DIN = 128
CW = 16
BLK = 160
SUB = 80
NBUF = 3

_ZACC = np.zeros((NSEG, DIN), np.float32)


def _sc_segment_sums(x, b2, bp, zacc):
    n = x.shape[0]
    nblk = n // BLK
    npk = bp.shape[0]
    nstream = NW - 1
    kmax = (nblk + nstream - 1) // nstream
    nvec = NSEG // CW
    steps = max(1, (n - 1).bit_length())
    mesh = plsc.VectorSubcoreMesh(
        core_axis_name="c", subcore_axis_name="s",
        num_cores=NC, num_subcores=NS)

    @functools.partial(
        pl.kernel,
        out_type=(
            jax.ShapeDtypeStruct((NC, NSEG, DIN), jnp.float32),
            jax.ShapeDtypeStruct((NSEG,), jnp.float32),
        ),
        mesh=mesh,
        compiler_params=pltpu.CompilerParams(needs_layout_passes=False),
        scratch_types=[
            pltpu.VMEM((NBUF, BLK, DIN), jnp.float32),
            pltpu.VMEM((NBUF, BLK // SUB, SUB), jnp.int32),
            pltpu.VMEM((npk,), jnp.int32),
            pltpu.VMEM((NSEG + 2 * CW,), jnp.int32),
            pltpu.VMEM((NSEG,), jnp.float32),
            pltpu.VMEM_SHARED((NSEG, DIN), jnp.float32),
            pltpu.SemaphoreType.DMA,
            pltpu.SemaphoreType.DMA,
        ],
    )
    def sc_kernel(x_hbm, b2_hbm, bp_hbm, zacc_hbm, pacc_hbm, cnt_hbm,
                  xbuf, idxbuf, bpk, ubuf, cbuf, acc, semx, semi):
        c = lax.axis_index("c")
        s = lax.axis_index("s")
        wid = s * NC + c

        @pl.when(s == 0)
        def _init():
            pltpu.sync_copy(zacc_hbm, acc)

        plsc.subcore_barrier()

        @pl.when(wid < nstream)
        def _stream():
            def start_fetch(i, p):
                blk = wid + i * nstream

                @pl.when(blk < nblk)
                def _():
                    pltpu.async_copy(
                        x_hbm.at[pl.ds(blk * BLK, BLK), :], xbuf.at[p], semx)
                    pltpu.async_copy(
                        b2_hbm.at[pl.ds(blk * (BLK // SUB), BLK // SUB), :],
                        idxbuf.at[p], semi)

            for j in range(NBUF - 1):
                start_fetch(j, j)

            def body(i, carry):
                p = lax.rem(i, NBUF)
                start_fetch(i + (NBUF - 1), lax.rem(i + (NBUF - 1), NBUF))
                blk = wid + i * nstream

                @pl.when(blk < nblk)
                def _():
                    pltpu.make_async_copy(
                        x_hbm.at[pl.ds(blk * BLK, BLK), :], xbuf.at[p],
                        semx).wait()
                    pltpu.make_async_copy(
                        b2_hbm.at[pl.ds(blk * (BLK // SUB), BLK // SUB), :],
                        idxbuf.at[p], semi).wait()
                    for q in range(BLK // SUB):
                        pltpu.sync_copy(
                            xbuf.at[p, pl.ds(q * SUB, SUB), :],
                            acc.at[idxbuf.at[p, q]], add=True)

                return carry

            lax.fori_loop(0, kmax, body, 0)

        @pl.when(wid == NW - 1)
        def _counts():
            pltpu.sync_copy(bp_hbm, bpk)
            zed = jnp.zeros((CW,), jnp.int32)
            ubuf[pl.ds(0, CW)] = zed

            def search(v, carry):
                t = lax.broadcasted_iota(jnp.int32, (CW,), 0) + v * CW + 1
                lo = jnp.zeros((CW,), jnp.int32)
                hi = jnp.full((CW,), n, jnp.int32)
                for _ in range(steps):
                    mid = lax.shift_right_logical(lo + hi, 1)
                    midc = jnp.minimum(mid, n - 1)
                    word = plsc.load_gather(
                        bpk, [lax.shift_right_logical(midc, 1)])
                    sh = lax.shift_left(jnp.bitwise_and(midc, 1), 4)
                    bv = jnp.bitwise_and(
                        lax.shift_right_logical(word, sh), 0xFFFF)
                    p = bv < t
                    lo = jnp.where(p, mid + 1, lo)
                    hi = jnp.where(p, hi, mid)
                ubuf[pl.ds(CW + v * CW, CW)] = hi
                return carry

            lax.fori_loop(0, nvec, search, 0)

            def diff(v, carry):
                a = ubuf[pl.ds(CW + v * CW, CW)]
                bb = ubuf[pl.ds(CW - 1 + v * CW, CW)]
                cbuf[pl.ds(v * CW, CW)] = (a - bb).astype(jnp.float32)
                return carry

            lax.fori_loop(0, nvec, diff, 0)
            pltpu.sync_copy(cbuf, cnt_hbm)

        plsc.subcore_barrier()

        @pl.when(s == 0)
        def _flush():
            pltpu.sync_copy(acc, pacc_hbm.at[c])

    return sc_kernel(x, b2, bp, zacc)


def _tc_finish(pacc, cnt, W, b):
    def body(pacc_ref, cnt_ref, w_ref, b_ref, out_ref):
        ssum = pacc_ref[0] + pacc_ref[1]
        cnts = cnt_ref[...].reshape(NSEG, 1)
        bias = b_ref[...].reshape(1, DIN)
        out_ref[...] = lax.dot_general(
            ssum, w_ref[...], (((1,), (1,)), ((), ())),
            preferred_element_type=jnp.float32) + cnts * bias

    return pl.pallas_call(
        body,
        out_shape=jax.ShapeDtypeStruct((NSEG, DIN), jnp.float32),
    )(pacc, cnt, W, b)


def kernel(x, edge_index, batch, W, b):
    del edge_index
    batch = batch.astype(jnp.int32)
    b2 = batch.reshape(x.shape[0] // SUB, SUB)
    bp = batch[0::2] | (batch[1::2] << 16)
    pacc, cnt = _sc_segment_sums(x, b2, bp, _ZACC)
    return _tc_finish(pacc, cnt, W, b)


# speedup vs baseline: 1.0451x; 1.0451x over previous
import functools

import numpy as np

import jax
import jax.numpy as jnp
from jax import lax
from jax.experimental import pallas as pl
from jax.experimental.pallas import tpu as pltpu
from jax.experimental.pallas import tpu_sc as plsc

NC = 2
NS = 16
NW = NC * NS

NSEG = 512
DIN = 128
CW = 16
BLK = 80
NBUF = 4

_ZACC = np.zeros((NSEG, DIN), np.float32)


def _sc_segment_sums(x, batch, bp, zacc):
    n = x.shape[0]
    nblk = n // BLK
    npk = bp.shape[0]
    nstream = NW - 1
    kmax = (nblk + nstream - 1) // nstream
    nvec = NSEG // CW
    steps = max(1, (n - 1).bit_length())
    mesh = plsc.VectorSubcoreMesh(
        core_axis_name="c", subcore_axis_name="s",
        num_cores=NC, num_subcores=NS)

    @functools.partial(
        pl.kernel,
        out_type=(
            jax.ShapeDtypeStruct((NC, NSEG, DIN), jnp.float32),
            jax.ShapeDtypeStruct((NSEG,), jnp.float32),
        ),
        mesh=mesh,
        compiler_params=pltpu.CompilerParams(needs_layout_passes=False),
        scratch_types=[
            pltpu.VMEM((NBUF, BLK, DIN), jnp.float32),
            pltpu.VMEM((NBUF, BLK), jnp.int32),
            pltpu.VMEM((npk,), jnp.int32),
            pltpu.VMEM((NSEG + 2 * CW,), jnp.int32),
            pltpu.VMEM((NSEG,), jnp.float32),
            pltpu.VMEM_SHARED((NSEG, DIN), jnp.float32),
            pltpu.SemaphoreType.DMA,
            pltpu.SemaphoreType.DMA,
        ],
    )
    def sc_kernel(x_hbm, b_hbm, bp_hbm, zacc_hbm, pacc_hbm, cnt_hbm,
                  xbuf, idxbuf, bpk, ubuf, cbuf, acc, semx, semi):
        c = lax.axis_index("c")
        s = lax.axis_index("s")
        wid = s * NC + c

        @pl.when(s == 0)
        def _init():
            pltpu.sync_copy(zacc_hbm, acc)

        plsc.subcore_barrier()

        @pl.when(wid < nstream)
        def _stream():
            def start_fetch(i, p):
                blk = wid + i * nstream

                @pl.when(blk < nblk)
                def _():
                    pltpu.async_copy(
                        x_hbm.at[pl.ds(blk * BLK, BLK), :], xbuf.at[p], semx)
                    pltpu.async_copy(
                        b_hbm.at[pl.ds(blk * BLK, BLK)], idxbuf.at[p], semi)

            for j in range(NBUF - 1):
                start_fetch(j, j)

            def body(i, carry):
                p = lax.rem(i, NBUF)
                start_fetch(i + (NBUF - 1), lax.rem(i + (NBUF - 1), NBUF))
                blk = wid + i * nstream

                @pl.when(blk < nblk)
                def _():
                    pltpu.make_async_copy(
                        x_hbm.at[pl.ds(blk * BLK, BLK), :], xbuf.at[p],
                        semx).wait()
                    pltpu.make_async_copy(
                        b_hbm.at[pl.ds(blk * BLK, BLK)], idxbuf.at[p],
                        semi).wait()
                    pltpu.sync_copy(xbuf.at[p], acc.at[idxbuf.at[p]],
                                    add=True)

                return carry

            lax.fori_loop(0, kmax, body, 0)

        @pl.when(wid == NW - 1)
        def _counts():
            pltpu.sync_copy(bp_hbm, bpk)
            zed = jnp.zeros((CW,), jnp.int32)
            ubuf[pl.ds(0, CW)] = zed

            def search(v, carry):
                t = lax.broadcasted_iota(jnp.int32, (CW,), 0) + v * CW + 1
                lo = jnp.zeros((CW,), jnp.int32)
                hi = jnp.full((CW,), n, jnp.int32)
                for _ in range(steps):
                    mid = lax.shift_right_logical(lo + hi, 1)
                    midc = jnp.minimum(mid, n - 1)
                    word = plsc.load_gather(
                        bpk, [lax.shift_right_logical(midc, 1)])
                    sh = lax.shift_left(jnp.bitwise_and(midc, 1), 4)
                    bv = jnp.bitwise_and(
                        lax.shift_right_logical(word, sh), 0xFFFF)
                    p = bv < t
                    lo = jnp.where(p, mid + 1, lo)
                    hi = jnp.where(p, hi, mid)
                ubuf[pl.ds(CW + v * CW, CW)] = hi
                return carry

            lax.fori_loop(0, nvec, search, 0)

            def diff(v, carry):
                a = ubuf[pl.ds(CW + v * CW, CW)]
                bb = ubuf[pl.ds(CW - 1 + v * CW, CW)]
                cbuf[pl.ds(v * CW, CW)] = (a - bb).astype(jnp.float32)
                return carry

            lax.fori_loop(0, nvec, diff, 0)
            pltpu.sync_copy(cbuf, cnt_hbm)

        plsc.subcore_barrier()

        @pl.when(s == 0)
        def _flush():
            pltpu.sync_copy(acc, pacc_hbm.at[c])

    return sc_kernel(x, batch, bp, zacc)


def _tc_finish(pacc, cnt, W, b):
    def body(pacc_ref, cnt_ref, w_ref, b_ref, out_ref):
        ssum = pacc_ref[0] + pacc_ref[1]
        cnts = cnt_ref[...].reshape(NSEG, 1)
        bias = b_ref[...].reshape(1, DIN)
        out_ref[...] = lax.dot_general(
            ssum, w_ref[...], (((1,), (1,)), ((), ())),
            preferred_element_type=jnp.float32) + cnts * bias

    return pl.pallas_call(
        body,
        out_shape=jax.ShapeDtypeStruct((NSEG, DIN), jnp.float32),
    )(pacc, cnt, W, b)


def kernel(x, edge_index, batch, W, b):
    del edge_index
    batch = batch.astype(jnp.int32)
    bp = batch[0::2] | (batch[1::2] << 16)
    pacc, cnt = _sc_segment_sums(x, batch, bp, _ZACC)
    return _tc_finish(pacc, cnt, W, b)


# speedup vs baseline: 1.2731x; 1.2181x over previous
import functools

import numpy as np

import jax
import jax.numpy as jnp
from jax import lax
from jax.experimental import pallas as pl
from jax.experimental.pallas import tpu as pltpu
from jax.experimental.pallas import tpu_sc as plsc

NC = 2
NS = 16
NW = NC * NS

NSEG = 512
DIN = 128
CW = 16
BLK = 80
NBUF = 2

_ZACC = np.zeros((NSEG, DIN), np.float32)


def _sc_segment_sums(x, batch, zacc):
    n = x.shape[0]
    nblk = n // BLK
    nstream = NW - 1
    kmax = (nblk + nstream - 1) // nstream
    nvec = NSEG // CW
    steps = max(1, (n - 1).bit_length())
    mesh = plsc.VectorSubcoreMesh(
        core_axis_name="c", subcore_axis_name="s",
        num_cores=NC, num_subcores=NS)

    @functools.partial(
        pl.kernel,
        out_type=(
            jax.ShapeDtypeStruct((NC, NSEG, DIN), jnp.float32),
            jax.ShapeDtypeStruct((NSEG,), jnp.float32),
        ),
        mesh=mesh,
        compiler_params=pltpu.CompilerParams(needs_layout_passes=False),
        scratch_types=[
            pltpu.VMEM((NBUF, BLK, DIN), jnp.float32),
            pltpu.VMEM((NBUF, BLK), jnp.int32),
            pltpu.VMEM((n,), jnp.int32),
            pltpu.VMEM((NSEG + 2 * CW,), jnp.int32),
            pltpu.VMEM((NSEG,), jnp.float32),
            pltpu.VMEM_SHARED((NSEG, DIN), jnp.float32),
            pltpu.SemaphoreType.DMA,
            pltpu.SemaphoreType.DMA,
        ],
    )
    def sc_kernel(x_hbm, b_hbm, zacc_hbm, pacc_hbm, cnt_hbm,
                  xbuf, idxbuf, bfull, ubuf, cbuf, acc, semx, semi):
        c = lax.axis_index("c")
        s = lax.axis_index("s")
        wid = s * NC + c

        rows = NSEG // NS
        pltpu.sync_copy(zacc_hbm.at[pl.ds(s * rows, rows), :],
                        acc.at[pl.ds(s * rows, rows), :])
        plsc.subcore_barrier()

        @pl.when(wid < nstream)
        def _stream():
            def start_fetch(i, p):
                blk = wid + i * nstream

                @pl.when(blk < nblk)
                def _():
                    pltpu.async_copy(
                        x_hbm.at[pl.ds(blk * BLK, BLK), :], xbuf.at[p], semx)
                    pltpu.async_copy(
                        b_hbm.at[pl.ds(blk * BLK, BLK)], idxbuf.at[p], semi)

            for j in range(NBUF - 1):
                start_fetch(j, j)

            def body(i, carry):
                p = lax.rem(i, NBUF)
                start_fetch(i + (NBUF - 1), lax.rem(i + (NBUF - 1), NBUF))
                blk = wid + i * nstream

                @pl.when(blk < nblk)
                def _():
                    pltpu.make_async_copy(
                        x_hbm.at[pl.ds(blk * BLK, BLK), :], xbuf.at[p],
                        semx).wait()
                    pltpu.make_async_copy(
                        b_hbm.at[pl.ds(blk * BLK, BLK)], idxbuf.at[p],
                        semi).wait()
                    pltpu.sync_copy(xbuf.at[p], acc.at[idxbuf.at[p]],
                                    add=True)

                return carry

            lax.fori_loop(0, kmax, body, 0)

        @pl.when(wid == NW - 1)
        def _counts():
            pltpu.sync_copy(b_hbm, bfull)
            zed = jnp.zeros((CW,), jnp.int32)
            ubuf[pl.ds(0, CW)] = zed

            def search(v, carry):
                t = lax.broadcasted_iota(jnp.int32, (CW,), 0) + v * CW + 1
                lo = jnp.zeros((CW,), jnp.int32)
                hi = jnp.full((CW,), n, jnp.int32)
                for _ in range(steps):
                    mid = lax.shift_right_logical(lo + hi, 1)
                    bv = plsc.load_gather(
                        bfull, [jnp.minimum(mid, n - 1)])
                    p = bv < t
                    lo = jnp.where(p, mid + 1, lo)
                    hi = jnp.where(p, hi, mid)
                ubuf[pl.ds(CW + v * CW, CW)] = hi
                return carry

            lax.fori_loop(0, nvec, search, 0)

            def diff(v, carry):
                a = ubuf[pl.ds(CW + v * CW, CW)]
                bb = ubuf[pl.ds(CW - 1 + v * CW, CW)]
                cbuf[pl.ds(v * CW, CW)] = (a - bb).astype(jnp.float32)
                return carry

            lax.fori_loop(0, nvec, diff, 0)
            pltpu.sync_copy(cbuf, cnt_hbm)

        plsc.subcore_barrier()

        pltpu.sync_copy(acc.at[pl.ds(s * rows, rows), :],
                        pacc_hbm.at[c, pl.ds(s * rows, rows), :])

    return sc_kernel(x, batch, zacc)


def _tc_finish(pacc, cnt, W, b):
    def body(pacc_ref, cnt_ref, w_ref, b_ref, out_ref):
        ssum = pacc_ref[0] + pacc_ref[1]
        cnts = cnt_ref[...].reshape(NSEG, 1)
        bias = b_ref[...].reshape(1, DIN)
        out_ref[...] = lax.dot_general(
            ssum, w_ref[...], (((1,), (1,)), ((), ())),
            preferred_element_type=jnp.float32) + cnts * bias

    return pl.pallas_call(
        body,
        out_shape=jax.ShapeDtypeStruct((NSEG, DIN), jnp.float32),
    )(pacc, cnt, W, b)


def kernel(x, edge_index, batch, W, b):
    del edge_index
    batch = batch.astype(jnp.int32)
    pacc, cnt = _sc_segment_sums(x, batch, _ZACC)
    return _tc_finish(pacc, cnt, W, b)
